# baseline (speedup 1.0000x reference)
import functools

import jax
import jax.numpy as jnp
from jax import lax
from jax.experimental import pallas as pl
from jax.experimental.pallas import tpu as pltpu
from jax.experimental.pallas import tpu_sc as plsc

N = 50000
D = 256
E = 150000

NC = 2
NS = 16
LANES = 16

CHUNK_W = 32
N_CHUNKS = D // CHUNK_W
EB = 64
EPT_B = 148
EPT = EPT_B * EB
EPAD = NS * EPT
NBUF = 4

ZROWS = 68
ZREP = 46
EXP_F = ZREP * ZROWS
EXP_L = N - 15 * EXP_F
ACC_R = NS * EXP_F


def _sc_body(h2, rows, cols,
             agg_out, deg_out,
             acc, rowb, colb, b0, b1, b2, b3, zbuf,
             g0, g1, g2, g3, s0, s1, s2, s3, zs):
    sid = lax.axis_index("s")
    bufs = [b0, b1, b2, b3]
    gsem = [g0, g1, g2, g3]
    ssem = [s0, s1, s2, s3]

    def fill(ref, nrows, val):
        def fb(i, carry):
            j = i // 2
            k = i % 2
            ref[j, pl.ds(k * LANES, LANES)] = jnp.full((LANES,), val,
                                                       jnp.float32)
            return carry
        lax.fori_loop(0, nrows * 2, fb, 0)

    fill(zbuf, ZROWS, 0.0)

    def zslice(i):
        return acc.at[pl.ds((sid * ZREP + i) * ZROWS, ZROWS)]

    def zero_start():
        def zf(i, carry):
            pltpu.async_copy(zbuf, zslice(i), zs)
            return carry
        lax.fori_loop(0, ZREP, zf, 0)

    def zero_drain():
        def zd(i, carry):
            pltpu.make_async_copy(zbuf, zslice(i), zs).wait()
            return carry
        lax.fori_loop(0, ZREP, zd, 0)

    def export_rows(make_dst):
        base = sid * EXP_F

        @pl.when(sid < NS - 1)
        def _():
            pltpu.sync_copy(acc.at[pl.ds(base, EXP_F)], make_dst(base, EXP_F))

        @pl.when(sid == NS - 1)
        def _():
            pltpu.sync_copy(acc.at[pl.ds(base, EXP_L)], make_dst(base, EXP_L))

    VPB = EB // LANES

    if True:
        zero_start()
        pltpu.sync_copy(rows.at[sid], rowb)
        pltpu.sync_copy(cols.at[sid], colb)
        fill(b0, EB, 1.0)
        zero_drain()
        plsc.subcore_barrier()

        def degf(j, carry):
            pltpu.async_copy(b0, acc.at[colb.at[j]], s0, add=True)
            return carry
        lax.fori_loop(0, EPT_B, degf, 0)

        def degd(j, carry):
            pltpu.make_async_copy(b0, acc.at[colb.at[j]], s0).wait()
            return carry
        lax.fori_loop(0, EPT_B, degd, 0)
        plsc.subcore_barrier()
        export_rows(lambda b, n: deg_out.at[pl.ds(b, n)])
        zero_start()

        def mul8(i, carry):
            j = i // VPB
            k = i % VPB
            sl = pl.ds(k * LANES, LANES)
            rowb[j, sl] = rowb[j, sl] * 8
            return carry
        lax.fori_loop(0, EPT_B * VPB, mul8, 0)
        zero_drain()
        plsc.subcore_barrier()

        for c in range(N_CHUNKS):
            for b in range(NBUF):
                pltpu.async_copy(h2.at[rowb.at[b]], bufs[b], gsem[b])

            def outer(t, carry):
                for b in range(NBUF):
                    j = t * NBUF + b
                    pltpu.make_async_copy(h2.at[rowb.at[j]], bufs[b],
                                          gsem[b]).wait()
                    pltpu.async_copy(bufs[b], acc.at[colb.at[j]], ssem[b],
                                     add=True)
                    pltpu.make_async_copy(bufs[b], acc.at[colb.at[j]],
                                          ssem[b]).wait()

                    @pl.when(j + NBUF < EPT_B)
                    def _():
                        pltpu.async_copy(h2.at[rowb.at[j + NBUF]], bufs[b],
                                         gsem[b])
                return carry
            lax.fori_loop(0, EPT_B // NBUF, outer, 0)
            plsc.subcore_barrier()
            export_rows(lambda b, n: agg_out.at[c, pl.ds(b, n)])

            if c < N_CHUNKS - 1:
                zero_start()

                def bump(i, carry):
                    j = i // VPB
                    k = i % VPB
                    sl = pl.ds(k * LANES, LANES)
                    rowb[j, sl] = rowb[j, sl] + 1
                    return carry
                lax.fori_loop(0, EPT_B * VPB, bump, 0)
                zero_drain()
                plsc.subcore_barrier()


_sc_agg_rel = pl.kernel(
    _sc_body,
    out_type=(
        jax.ShapeDtypeStruct((N_CHUNKS, N, CHUNK_W), jnp.float32),
        jax.ShapeDtypeStruct((N, CHUNK_W), jnp.float32),
    ),
    mesh=plsc.VectorSubcoreMesh(core_axis_name="c", subcore_axis_name="s",
                                num_cores=1),
    scratch_types=[
        pltpu.VMEM_SHARED((ACC_R, CHUNK_W), jnp.float32),
        pltpu.VMEM((EPT_B, EB), jnp.int32),
        pltpu.VMEM((EPT_B, EB), jnp.int32),
        pltpu.VMEM((EB, CHUNK_W), jnp.float32),
        pltpu.VMEM((EB, CHUNK_W), jnp.float32),
        pltpu.VMEM((EB, CHUNK_W), jnp.float32),
        pltpu.VMEM((EB, CHUNK_W), jnp.float32),
        pltpu.VMEM((ZROWS, CHUNK_W), jnp.float32),
        pltpu.SemaphoreType.DMA,
        pltpu.SemaphoreType.DMA,
        pltpu.SemaphoreType.DMA,
        pltpu.SemaphoreType.DMA,
        pltpu.SemaphoreType.DMA,
        pltpu.SemaphoreType.DMA,
        pltpu.SemaphoreType.DMA,
        pltpu.SemaphoreType.DMA,
        pltpu.SemaphoreType.DMA,
    ],
    compiler_params=pltpu.CompilerParams(use_tc_tiling_on_sc=False),
)


def _fuse_body(hp, hc, aggp, aggc, degp, degc, wsp, wsc, wpc, wcp, bb,
               outp, outc):
    mm = functools.partial(
        lax.dot_general,
        dimension_numbers=(((1,), (1,)), ((), ())),
        preferred_element_type=jnp.float32)

    dp = degp[:, 0:1]
    invp = jnp.where(dp > 0, 1.0 / dp, 0.0)
    ap = jnp.concatenate([aggp[c] for c in range(N_CHUNKS)], axis=1)
    op = mm(hp[...], wsp[...]) + mm(ap * invp, wcp[...]) + bb[0]
    outp[...] = jnp.maximum(op, 0.0)

    dc = degc[:, 0:1]
    invc = jnp.where(dc > 0, 1.0 / dc, 0.0)
    ac = jnp.concatenate([aggc[c] for c in range(N_CHUNKS)], axis=1)
    oc = mm(hc[...], wsc[...]) + mm(ac * invc, wpc[...]) + bb[1]
    outc[...] = jnp.maximum(oc, 0.0)


ROW_T = 400


def _fuse(hp, hc, aggp, aggc, degp, degc, wsp, wsc, wpc, wcp, bb):
    row_spec = pl.BlockSpec((ROW_T, D), lambda i: (i, 0))
    agg_spec = pl.BlockSpec((N_CHUNKS, ROW_T, CHUNK_W), lambda i: (0, i, 0))
    deg_spec = pl.BlockSpec((ROW_T, CHUNK_W), lambda i: (i, 0))
    w_spec = pl.BlockSpec((D, D), lambda i: (0, 0))
    b_spec = pl.BlockSpec((8, D), lambda i: (0, 0))
    return pl.pallas_call(
        _fuse_body,
        grid=(N // ROW_T,),
        in_specs=[row_spec, row_spec, agg_spec, agg_spec,
                  deg_spec, deg_spec, w_spec, w_spec, w_spec, w_spec,
                  b_spec],
        out_specs=[row_spec, row_spec],
        out_shape=[jax.ShapeDtypeStruct((N, D), jnp.float32),
                   jax.ShapeDtypeStruct((N, D), jnp.float32)],
    )(hp, hc, aggp, aggc, degp, degc, wsp, wsc, wpc, wcp, bb)


def _prep_edges(ei):
    row = jnp.pad(ei[0], (0, EPAD - E))
    col = jnp.pad(ei[1], (0, EPAD - E),
                  constant_values=N)
    return (row.reshape(NS, EPT_B, EB), col.reshape(NS, EPT_B, EB))


def kernel(h_project, h_company,
           edge_index_project_royalty_company, edge_index_company_owns_project,
           W_self_project, b_self_project, W_self_company, b_self_company,
           W_rel_pc, b_rel_pc, W_rel_cp, b_rel_cp):
    hp2 = h_project.reshape(N_CHUNKS * N, CHUNK_W)
    hc2 = h_company.reshape(N_CHUNKS * N, CHUNK_W)
    rows_pc, cols_pc = _prep_edges(edge_index_project_royalty_company)
    rows_cp, cols_cp = _prep_edges(edge_index_company_owns_project)

    agg_c, deg_c = _sc_agg_rel(hp2, rows_pc, cols_pc)
    agg_p, deg_p = _sc_agg_rel(hc2, rows_cp, cols_cp)

    bb = jnp.zeros((8, D), jnp.float32)
    bb = bb.at[0].set(b_self_project + b_rel_cp)
    bb = bb.at[1].set(b_self_company + b_rel_pc)

    out_p, out_c = _fuse(h_project, h_company, agg_p, agg_c, deg_p, deg_c,
                         W_self_project, W_self_company, W_rel_pc, W_rel_cp,
                         bb)
    return (out_p, out_c)


# speedup vs baseline: 1.7082x; 1.7082x over previous
import functools

import jax
import jax.numpy as jnp
from jax import lax
from jax.experimental import pallas as pl
from jax.experimental.pallas import tpu as pltpu
from jax.experimental.pallas import tpu_sc as plsc

N = 50000
D = 256
E = 150000

NC = 2
NS = 16
LANES = 16

CHUNK_W = 64
N_CHUNKS = D // CHUNK_W
EB = 64
EPT_B = 148
EPT = EPT_B * EB
EPAD = NS * EPT
NBUF = 4

ZROWS = 98
ZREP = 32
EXP_F = ZREP * ZROWS
EXP_L = N - 15 * EXP_F
ACC_R = NS * EXP_F

BF16 = jnp.bfloat16


def _sc_body(hp2, hc2, rows_pc, cols_pc, rows_cp, cols_cp,
             agg_c, deg_c, agg_p, deg_p,
             acc, rowb, colb, b0, b1, b2, b3, zbuf,
             g0, g1, g2, g3, s0, s1, s2, s3, zs):
    cid = lax.axis_index("c")
    sid = lax.axis_index("s")
    bufs = [b0, b1, b2, b3]
    gsem = [g0, g1, g2, g3]
    ssem = [s0, s1, s2, s3]

    def fill(ref, nrows, val):
        def fb(i, carry):
            j = i // 2
            k = i % 2
            ref[j, pl.ds(k * 2 * LANES, 2 * LANES)] = jnp.full(
                (2 * LANES,), val, BF16)
            return carry
        lax.fori_loop(0, nrows * 2, fb, 0)

    fill(zbuf, ZROWS, 0.0)

    def zslice(i):
        return acc.at[pl.ds((sid * ZREP + i) * ZROWS, ZROWS)]

    def zero_start():
        def zf(i, carry):
            pltpu.async_copy(zbuf, zslice(i), zs)
            return carry
        lax.fori_loop(0, ZREP, zf, 0)

    def zero_drain():
        def zd(i, carry):
            pltpu.make_async_copy(zbuf, zslice(i), zs).wait()
            return carry
        lax.fori_loop(0, ZREP, zd, 0)

    def export_rows(make_dst):
        base = sid * EXP_F

        @pl.when(sid < NS - 1)
        def _():
            pltpu.sync_copy(acc.at[pl.ds(base, EXP_F)], make_dst(base, EXP_F))

        @pl.when(sid == NS - 1)
        def _():
            pltpu.sync_copy(acc.at[pl.ds(base, EXP_L)], make_dst(base, EXP_L))

    VPB = EB // LANES

    def do_rel(h2, rows, cols, agg_out, deg_out):
        zero_start()
        pltpu.sync_copy(rows.at[sid], rowb)
        pltpu.sync_copy(cols.at[sid], colb)
        fill(b0, EB, 1.0)
        zero_drain()
        plsc.subcore_barrier()

        def degf(j, carry):
            pltpu.async_copy(b0, acc.at[colb.at[j]], s0, add=True)
            return carry
        lax.fori_loop(0, EPT_B, degf, 0)

        def degd(j, carry):
            pltpu.make_async_copy(b0, acc.at[colb.at[j]], s0).wait()
            return carry
        lax.fori_loop(0, EPT_B, degd, 0)
        plsc.subcore_barrier()
        export_rows(lambda b, n: deg_out.at[pl.ds(b, n)])
        zero_start()

        def mulc(i, carry):
            j = i // VPB
            k = i % VPB
            sl = pl.ds(k * LANES, LANES)
            rowb[j, sl] = rowb[j, sl] * N_CHUNKS
            return carry
        lax.fori_loop(0, EPT_B * VPB, mulc, 0)
        zero_drain()
        plsc.subcore_barrier()

        for c in range(N_CHUNKS):
            for b in range(NBUF):
                pltpu.async_copy(h2.at[rowb.at[b]], bufs[b], gsem[b])

            def outer(t, carry):
                for b in range(NBUF):
                    j = t * NBUF + b
                    pltpu.make_async_copy(h2.at[rowb.at[j]], bufs[b],
                                          gsem[b]).wait()
                    pltpu.async_copy(bufs[b], acc.at[colb.at[j]], ssem[b],
                                     add=True)
                    pltpu.make_async_copy(bufs[b], acc.at[colb.at[j]],
                                          ssem[b]).wait()

                    @pl.when(j + NBUF < EPT_B)
                    def _():
                        pltpu.async_copy(h2.at[rowb.at[j + NBUF]], bufs[b],
                                         gsem[b])
                return carry
            lax.fori_loop(0, EPT_B // NBUF, outer, 0)
            plsc.subcore_barrier()
            export_rows(lambda b, n: agg_out.at[c, pl.ds(b, n)])

            if c < N_CHUNKS - 1:
                zero_start()

                def bump(i, carry):
                    j = i // VPB
                    k = i % VPB
                    sl = pl.ds(k * LANES, LANES)
                    rowb[j, sl] = rowb[j, sl] + 1
                    return carry
                lax.fori_loop(0, EPT_B * VPB, bump, 0)
                zero_drain()
                plsc.subcore_barrier()

    @pl.when(cid == 0)
    def _():
        do_rel(hp2, rows_pc, cols_pc, agg_c, deg_c)

    @pl.when(cid == 1)
    def _():
        do_rel(hc2, rows_cp, cols_cp, agg_p, deg_p)


_sc_agg = pl.kernel(
    _sc_body,
    out_type=(
        jax.ShapeDtypeStruct((N_CHUNKS, N, CHUNK_W), BF16),
        jax.ShapeDtypeStruct((N, CHUNK_W), BF16),
        jax.ShapeDtypeStruct((N_CHUNKS, N, CHUNK_W), BF16),
        jax.ShapeDtypeStruct((N, CHUNK_W), BF16),
    ),
    mesh=plsc.VectorSubcoreMesh(core_axis_name="c", subcore_axis_name="s"),
    scratch_types=[
        pltpu.VMEM_SHARED((ACC_R, CHUNK_W), BF16),
        pltpu.VMEM((EPT_B, EB), jnp.int32),
        pltpu.VMEM((EPT_B, EB), jnp.int32),
        pltpu.VMEM((EB, CHUNK_W), BF16),
        pltpu.VMEM((EB, CHUNK_W), BF16),
        pltpu.VMEM((EB, CHUNK_W), BF16),
        pltpu.VMEM((EB, CHUNK_W), BF16),
        pltpu.VMEM((ZROWS, CHUNK_W), BF16),
        pltpu.SemaphoreType.DMA,
        pltpu.SemaphoreType.DMA,
        pltpu.SemaphoreType.DMA,
        pltpu.SemaphoreType.DMA,
        pltpu.SemaphoreType.DMA,
        pltpu.SemaphoreType.DMA,
        pltpu.SemaphoreType.DMA,
        pltpu.SemaphoreType.DMA,
        pltpu.SemaphoreType.DMA,
    ],
    compiler_params=pltpu.CompilerParams(use_tc_tiling_on_sc=False),
)


def _fuse_body(hp, hc, aggp, aggc, degp, degc, wsp, wsc, wpc, wcp, bb,
               outp, outc):
    mm = functools.partial(
        lax.dot_general,
        dimension_numbers=(((1,), (1,)), ((), ())),
        preferred_element_type=jnp.float32)

    dp = degp[:, 0:1].astype(jnp.float32)
    invp = jnp.where(dp > 0, 1.0 / dp, 0.0)
    ap = jnp.concatenate([aggp[c] for c in range(N_CHUNKS)],
                         axis=1).astype(jnp.float32)
    op = mm(hp[...], wsp[...]) + mm(ap * invp, wcp[...]) + bb[0]
    outp[...] = jnp.maximum(op, 0.0)

    dc = degc[:, 0:1].astype(jnp.float32)
    invc = jnp.where(dc > 0, 1.0 / dc, 0.0)
    ac = jnp.concatenate([aggc[c] for c in range(N_CHUNKS)],
                         axis=1).astype(jnp.float32)
    oc = mm(hc[...], wsc[...]) + mm(ac * invc, wpc[...]) + bb[1]
    outc[...] = jnp.maximum(oc, 0.0)


ROW_T = 400


def _fuse(hp, hc, aggp, aggc, degp, degc, wsp, wsc, wpc, wcp, bb):
    row_spec = pl.BlockSpec((ROW_T, D), lambda i: (i, 0))
    agg_spec = pl.BlockSpec((N_CHUNKS, ROW_T, CHUNK_W), lambda i: (0, i, 0))
    deg_spec = pl.BlockSpec((ROW_T, CHUNK_W), lambda i: (i, 0))
    w_spec = pl.BlockSpec((D, D), lambda i: (0, 0))
    b_spec = pl.BlockSpec((8, D), lambda i: (0, 0))
    return pl.pallas_call(
        _fuse_body,
        grid=(N // ROW_T,),
        in_specs=[row_spec, row_spec, agg_spec, agg_spec,
                  deg_spec, deg_spec, w_spec, w_spec, w_spec, w_spec,
                  b_spec],
        out_specs=[row_spec, row_spec],
        out_shape=[jax.ShapeDtypeStruct((N, D), jnp.float32),
                   jax.ShapeDtypeStruct((N, D), jnp.float32)],
    )(hp, hc, aggp, aggc, degp, degc, wsp, wsc, wpc, wcp, bb)


def _prep_edges(ei):
    row = jnp.pad(ei[0], (0, EPAD - E))
    col = jnp.pad(ei[1], (0, EPAD - E),
                  constant_values=N)
    return (row.reshape(NS, EPT_B, EB), col.reshape(NS, EPT_B, EB))


def kernel(h_project, h_company,
           edge_index_project_royalty_company, edge_index_company_owns_project,
           W_self_project, b_self_project, W_self_company, b_self_company,
           W_rel_pc, b_rel_pc, W_rel_cp, b_rel_cp):
    hp2 = h_project.astype(BF16).reshape(N_CHUNKS * N, CHUNK_W)
    hc2 = h_company.astype(BF16).reshape(N_CHUNKS * N, CHUNK_W)
    rows_pc, cols_pc = _prep_edges(edge_index_project_royalty_company)
    rows_cp, cols_cp = _prep_edges(edge_index_company_owns_project)

    agg_c, deg_c, agg_p, deg_p = _sc_agg(hp2, hc2, rows_pc, cols_pc,
                                         rows_cp, cols_cp)

    bb = jnp.zeros((8, D), jnp.float32)
    bb = bb.at[0].set(b_self_project + b_rel_cp)
    bb = bb.at[1].set(b_self_company + b_rel_pc)

    out_p, out_c = _fuse(h_project, h_company, agg_p, agg_c, deg_p, deg_c,
                         W_self_project, W_self_company, W_rel_pc, W_rel_cp,
                         bb)
    return (out_p, out_c)
